# baseline (speedup 1.0000x reference)
import functools

import jax
import jax.numpy as jnp
from jax import lax
from jax.experimental import pallas as pl
from jax.experimental.pallas import tpu as pltpu
from jax.experimental.pallas import tpu_sc as plsc

P = 0.05
_LANES = 16
_NTILES = 32


def _drop16(base):
    n = base.astype(jnp.uint32) + lax.iota(jnp.uint32, 16)
    k1 = jnp.uint32(0)
    k2 = jnp.uint32(42)
    ks0, ks1, ks2 = k1, k2, k1 ^ k2 ^ jnp.uint32(0x1BD11BDA)
    rots = ((13, 15, 26, 6), (17, 29, 16, 24))
    kseq = ((ks1, ks2), (ks2, ks0), (ks0, ks1), (ks1, ks2), (ks2, ks0))
    x0 = jnp.zeros((16,), jnp.uint32) + ks0
    x1 = n + ks1
    for i in range(5):
        for r in rots[i % 2]:
            x0 = x0 + x1
            x1 = (x1 << jnp.uint32(r)) | (x1 >> jnp.uint32(32 - r))
            x1 = x0 ^ x1
        ka, kb = kseq[i]
        x0 = x0 + ka
        x1 = x1 + kb + jnp.uint32(i + 1)
    bits = x0 ^ x1
    return (bits >> jnp.uint32(9)) < jnp.uint32(419431)


@functools.partial(jax.jit, static_argnames=("pad", "tpw"))
def _node_drop_masks(m, *, pad, tpw):
    mesh = plsc.VectorSubcoreMesh(core_axis_name="c", subcore_axis_name="s")

    @functools.partial(
        pl.kernel,
        mesh=mesh,
        out_type=jax.ShapeDtypeStruct((3 * pad,), jnp.int32),
        scratch_types=[pltpu.VMEM((3 * tpw,), jnp.int32)],
    )
    def body(m_hbm, out_hbm, buf):
        wid = lax.axis_index("s") * 2 + lax.axis_index("c")
        base = pl.multiple_of(wid * tpw, 8)
        for k in range(3):
            pltpu.sync_copy(
                m_hbm.at[pl.ds(base + k * pad, tpw)],
                buf.at[pl.ds(k * tpw, tpw)],
            )
        for c in range(tpw // _LANES):
            off = c * _LANES
            drop = _drop16(base + off)
            zero = jnp.zeros((16,), jnp.int32)
            for k in range(3):
                sl = pl.ds(k * tpw + off, _LANES)
                buf[sl] = jnp.where(drop, zero, buf[sl])
        for k in range(3):
            pltpu.sync_copy(
                buf.at[pl.ds(k * tpw, tpw)],
                out_hbm.at[pl.ds(base + k * pad, tpw)],
            )

    return body(m)


def kernel(x, edge_index, y, train_mask, test_mask, val_mask):
    n = train_mask.shape[0]
    chunk = _NTILES * _LANES
    pad = ((n + chunk - 1) // chunk) * chunk
    tpw = pad // _NTILES
    m = jnp.concatenate(
        [
            jnp.pad(train_mask.astype(jnp.int32), (0, pad - n)),
            jnp.pad(test_mask.astype(jnp.int32), (0, pad - n)),
            jnp.pad(val_mask.astype(jnp.int32), (0, pad - n)),
        ]
    )
    out = _node_drop_masks(m, pad=pad, tpw=tpw)
    new_train = out[0:n].astype(jnp.bool_)
    new_test = out[pad:pad + n].astype(jnp.bool_)
    new_val = out[2 * pad:2 * pad + n].astype(jnp.bool_)
    return (x, edge_index, y, new_train, new_val, new_test)


# speedup vs baseline: 1.0431x; 1.0431x over previous
import functools

import jax
import jax.numpy as jnp
from jax import lax
from jax.experimental import pallas as pl
from jax.experimental.pallas import tpu as pltpu
from jax.experimental.pallas import tpu_sc as plsc

P = 0.05
_LANES = 16
_NTILES = 32


def _drop16(base):
    n = base.astype(jnp.uint32) + lax.iota(jnp.uint32, 16)
    k1 = jnp.uint32(0)
    k2 = jnp.uint32(42)
    ks0, ks1, ks2 = k1, k2, k1 ^ k2 ^ jnp.uint32(0x1BD11BDA)
    rots = ((13, 15, 26, 6), (17, 29, 16, 24))
    kseq = ((ks1, ks2), (ks2, ks0), (ks0, ks1), (ks1, ks2), (ks2, ks0))
    x0 = jnp.zeros((16,), jnp.uint32) + ks0
    x1 = n + ks1
    for i in range(5):
        for r in rots[i % 2]:
            x0 = x0 + x1
            x1 = (x1 << jnp.uint32(r)) | (x1 >> jnp.uint32(32 - r))
            x1 = x0 ^ x1
        ka, kb = kseq[i]
        x0 = x0 + ka
        x1 = x1 + kb + jnp.uint32(i + 1)
    bits = x0 ^ x1
    return (bits >> jnp.uint32(9)) < jnp.uint32(419431)


@functools.partial(jax.jit, static_argnames=("pad", "tpw"))
def _node_drop_masks(m, *, pad, tpw):
    mesh = plsc.VectorSubcoreMesh(core_axis_name="c", subcore_axis_name="s")

    @functools.partial(
        pl.kernel,
        mesh=mesh,
        out_type=jax.ShapeDtypeStruct((3 * pad,), jnp.int32),
        scratch_types=[pltpu.VMEM((3 * tpw,), jnp.int32)],
    )
    def body(m_hbm, out_hbm, buf):
        wid = lax.axis_index("s") * 2 + lax.axis_index("c")
        base = pl.multiple_of(wid * tpw, 8)
        for k in range(3):
            pltpu.sync_copy(
                m_hbm.at[pl.ds(base + k * pad, tpw)],
                buf.at[pl.ds(k * tpw, tpw)],
            )
        zero = jnp.zeros((16,), jnp.int32)

        def chunk(c, carry):
            off = c * _LANES
            drop = _drop16(base + off)
            for k in range(3):
                sl = pl.ds(k * tpw + off, _LANES)
                buf[sl] = jnp.where(drop, zero, buf[sl])
            return carry

        lax.fori_loop(0, tpw // _LANES, chunk, 0)
        for k in range(3):
            pltpu.sync_copy(
                buf.at[pl.ds(k * tpw, tpw)],
                out_hbm.at[pl.ds(base + k * pad, tpw)],
            )

    return body(m)


def kernel(x, edge_index, y, train_mask, test_mask, val_mask):
    n = train_mask.shape[0]
    chunk = _NTILES * _LANES
    pad = ((n + chunk - 1) // chunk) * chunk
    tpw = pad // _NTILES
    m = jnp.concatenate(
        [
            jnp.pad(train_mask.astype(jnp.int32), (0, pad - n)),
            jnp.pad(test_mask.astype(jnp.int32), (0, pad - n)),
            jnp.pad(val_mask.astype(jnp.int32), (0, pad - n)),
        ]
    )
    out = _node_drop_masks(m, pad=pad, tpw=tpw)
    new_train = out[0:n].astype(jnp.bool_)
    new_test = out[pad:pad + n].astype(jnp.bool_)
    new_val = out[2 * pad:2 * pad + n].astype(jnp.bool_)
    return (x, edge_index, y, new_train, new_val, new_test)
